# baseline (device time: 572390 ns/iter reference)
import jax
import jax.numpy as jnp
from jax import lax
from jax.experimental import pallas as pl
from jax.experimental.pallas import tpu as pltpu

N_CHUNKS = 8


def kernel(x):
    x16 = x.astype(jnp.bfloat16)
    m, n = x16.shape
    half = n // 2
    mc = m // N_CHUNKS

    def body(x_ref, out_ref, send_buf, pack_sems, local_sem, send_sems, recv_sems):
        my_p = lax.axis_index("x")
        my_y = lax.axis_index("y")
        my_z = lax.axis_index("z")
        peer = 1 - my_p

        local = pltpu.make_async_copy(
            x_ref.at[:, pl.ds(my_p * half, half)],
            out_ref.at[pl.ds(my_p * m, m), :],
            local_sem,
        )
        local.start()

        packs = []
        for k in range(N_CHUNKS):
            p = pltpu.make_async_copy(
                x_ref.at[pl.ds(k * mc, mc), pl.ds(peer * half, half)],
                send_buf.at[k],
                pack_sems.at[k],
            )
            p.start()
            packs.append(p)

        rdmas = []
        for k in range(N_CHUNKS):
            packs[k].wait()
            r = pltpu.make_async_remote_copy(
                src_ref=send_buf.at[k],
                dst_ref=out_ref.at[pl.ds(my_p * m + k * mc, mc), :],
                send_sem=send_sems.at[k],
                recv_sem=recv_sems.at[k],
                device_id=(peer, my_y, my_z),
                device_id_type=pl.DeviceIdType.MESH,
            )
            r.start()
            rdmas.append(r)

        local.wait()
        for r in rdmas:
            r.wait()

    return pl.pallas_call(
        body,
        out_shape=jax.ShapeDtypeStruct((2 * m, half), jnp.bfloat16),
        in_specs=[pl.BlockSpec(memory_space=pltpu.MemorySpace.HBM)],
        out_specs=pl.BlockSpec(memory_space=pltpu.MemorySpace.HBM),
        scratch_shapes=[
            pltpu.VMEM((N_CHUNKS, mc, half), jnp.bfloat16),
            pltpu.SemaphoreType.DMA((N_CHUNKS,)),
            pltpu.SemaphoreType.DMA,
            pltpu.SemaphoreType.DMA((N_CHUNKS,)),
            pltpu.SemaphoreType.DMA((N_CHUNKS,)),
        ],
    )(x16)


# device time: 213195 ns/iter; 2.6848x vs baseline; 2.6848x over previous
import jax
import jax.numpy as jnp
from jax import lax
from jax.experimental import pallas as pl
from jax.experimental.pallas import tpu as pltpu

N_CHUNKS = 16


def kernel(x):
    m, n = x.shape
    half = n // 2
    mc = m // N_CHUNKS

    def body(x_ref, out_ref, in_buf, send_buf, loc_buf,
             in_sems, out_sems, send_sems, recv_sems):
        my_p = lax.axis_index("x")
        my_y = lax.axis_index("y")
        my_z = lax.axis_index("z")
        peer = 1 - my_p

        def in_dma(k):
            return pltpu.make_async_copy(
                x_ref.at[pl.ds(k * mc, mc), :],
                in_buf.at[k % 2],
                in_sems.at[k % 2],
            )

        def out_dma(k):
            return pltpu.make_async_copy(
                loc_buf.at[k % 2],
                out_ref.at[pl.ds(my_p * m + k * mc, mc), :],
                out_sems.at[k % 2],
            )

        in_dma(0).start()
        rdmas = []
        for k in range(N_CHUNKS):
            in_dma(k).wait()
            if k + 1 < N_CHUNKS:
                in_dma(k + 1).start()

            if k >= 2:
                out_dma(k - 2).wait()

            @pl.when(my_p == 0)
            def _():
                loc_buf[k % 2] = in_buf[k % 2, :, :half].astype(jnp.bfloat16)
                send_buf[k] = in_buf[k % 2, :, half:].astype(jnp.bfloat16)

            @pl.when(my_p == 1)
            def _():
                loc_buf[k % 2] = in_buf[k % 2, :, half:].astype(jnp.bfloat16)
                send_buf[k] = in_buf[k % 2, :, :half].astype(jnp.bfloat16)

            out_dma(k).start()
            r = pltpu.make_async_remote_copy(
                src_ref=send_buf.at[k],
                dst_ref=out_ref.at[pl.ds(my_p * m + k * mc, mc), :],
                send_sem=send_sems.at[k],
                recv_sem=recv_sems.at[k],
                device_id=(peer, my_y, my_z),
                device_id_type=pl.DeviceIdType.MESH,
            )
            r.start()
            rdmas.append(r)

        out_dma(N_CHUNKS - 2).wait()
        out_dma(N_CHUNKS - 1).wait()
        for r in rdmas:
            r.wait()

    return pl.pallas_call(
        body,
        out_shape=jax.ShapeDtypeStruct((2 * m, half), jnp.bfloat16),
        in_specs=[pl.BlockSpec(memory_space=pltpu.MemorySpace.HBM)],
        out_specs=pl.BlockSpec(memory_space=pltpu.MemorySpace.HBM),
        scratch_shapes=[
            pltpu.VMEM((2, mc, n), jnp.float32),
            pltpu.VMEM((N_CHUNKS, mc, half), jnp.bfloat16),
            pltpu.VMEM((2, mc, half), jnp.bfloat16),
            pltpu.SemaphoreType.DMA((2,)),
            pltpu.SemaphoreType.DMA((2,)),
            pltpu.SemaphoreType.DMA((N_CHUNKS,)),
            pltpu.SemaphoreType.DMA((N_CHUNKS,)),
        ],
    )(x)
